# baseline (speedup 1.0000x reference)
import dataclasses
import functools
import jax
import jax.numpy as jnp
from jax import lax
from jax.experimental import pallas as pl
from jax.experimental.pallas import tpu as pltpu
from jax.experimental.pallas import tpu_sc as plsc

N_ROWS = 32768
D = 128
B = 16
L = 16
NVEC = D // L
NW = 32
NR = 496
PAY = NR - 8
MAXC = 3


def _sc_partial_means(x, batch_lengths):
    mesh = plsc.VectorSubcoreMesh(
        core_axis_name="c", subcore_axis_name="s", num_cores=2, num_subcores=16
    )
    cp = pltpu.CompilerParams()
    if "needs_layout_passes" in pltpu.CompilerParams.__dataclass_fields__:
        cp = dataclasses.replace(cp, needs_layout_passes=False)

    @functools.partial(
        pl.kernel,
        out_type=jax.ShapeDtypeStruct((NW, B, D), jnp.float32),
        mesh=mesh,
        scratch_types=[
            pltpu.VMEM((B,), jnp.int32),
            pltpu.VMEM((NR, D), jnp.float32),
            pltpu.VMEM((NR, D), jnp.float32),
            pltpu.VMEM((B, D), jnp.float32),
            pltpu.SemaphoreType.DMA,
            pltpu.SemaphoreType.DMA,
        ],
        compiler_params=cp,
    )
    def kern(x_hbm, len_hbm, out_hbm, len_vmem, buf0, buf1, part, sem0, sem1):
        c = lax.axis_index("c")
        s = lax.axis_index("s")
        w = c * 16 + s

        pltpu.sync_copy(len_hbm, len_vmem)
        lv = len_vmem[...]
        ends = plsc.cumsum(lv)
        lanes = lax.iota(jnp.int32, L)
        zeros_i = jnp.zeros((L,), jnp.int32)

        def lane(vec, i):
            return jnp.sum(jnp.where(lanes == i, vec, zeros_i))

        total = lane(ends, B - 1)
        seg_end = [lane(ends, i) for i in range(B)]
        seg_len = [lane(lv, i) for i in range(B)]

        rows_per_w = (total + NW - 1) // NW
        r0 = w * rows_per_w
        r1 = jnp.minimum(r0 + rows_per_w, total)
        cnt = jnp.maximum(r1 - r0, 0)

        zf = jnp.zeros((L,), jnp.float32)
        for i in range(B):
            for j in range(NVEC):
                part[i, pl.ds(L * j, L)] = zf

        bufs = [buf0, buf1, buf0]
        sems = [sem0, sem1, sem0]
        gstarts, aligneds, glens = [], [], []
        for k in range(MAXC):
            gstart = r0 + k * PAY
            aligned = jnp.minimum((gstart // 8) * 8, N_ROWS - NR)
            glen = jnp.minimum(PAY, r1 - gstart)
            gstarts.append(gstart)
            aligneds.append(aligned)
            glens.append(glen)

        def start_copy(k):
            pltpu.async_copy(
                x_hbm.at[pl.ds(aligneds[k], NR)], bufs[k], sems[k]
            )

        def wait_copy(k):
            pltpu.make_async_copy(
                x_hbm.at[pl.ds(aligneds[k], NR)], bufs[k], sems[k]
            ).wait()

        def process(k):
            buf = bufs[k]
            gstart, aligned, glen = gstarts[k], aligneds[k], glens[k]
            off = gstart - aligned
            gend = gstart + glen
            for i in range(B):
                seg_start = seg_end[i] - seg_len[i]
                lo = jnp.maximum(seg_start, gstart)
                hi = jnp.minimum(seg_end[i], gend)
                n = hi - lo

                @pl.when(n > 0)
                def _():
                    base = off + (lo - gstart)

                    def row_body(r, a):
                        q = base + r
                        return tuple(
                            a[j] + buf[q, pl.ds(L * j, L)] for j in range(NVEC)
                        )
                    accs = lax.fori_loop(
                        0, n, row_body,
                        tuple(zf for _ in range(NVEC)),
                    )
                    for j in range(NVEC):
                        part[i, pl.ds(L * j, L)] = (
                            part[i, pl.ds(L * j, L)] + accs[j]
                        )

        @pl.when(glens[0] > 0)
        def _():
            start_copy(0)

        @pl.when(glens[1] > 0)
        def _():
            start_copy(1)

        @pl.when(glens[0] > 0)
        def _():
            wait_copy(0)
            process(0)

        @pl.when(glens[2] > 0)
        def _():
            start_copy(2)

        @pl.when(glens[1] > 0)
        def _():
            wait_copy(1)
            process(1)

        @pl.when(glens[2] > 0)
        def _():
            wait_copy(2)
            process(2)

        for i in range(B):
            den = jnp.full((L,), seg_len[i], jnp.float32)
            for j in range(NVEC):
                part[i, pl.ds(L * j, L)] = part[i, pl.ds(L * j, L)] / den
        pltpu.sync_copy(part, out_hbm.at[w])

    return kern(x, batch_lengths)


def _combine_kernel(p_ref, o_ref):
    o_ref[...] = jnp.sum(p_ref[...], axis=0)


def kernel(x, batch_lengths):
    lens = batch_lengths.astype(jnp.int32)
    partials = _sc_partial_means(x, lens)
    return pl.pallas_call(
        _combine_kernel,
        out_shape=jax.ShapeDtypeStruct((B, D), jnp.float32),
    )(partials)


# speedup vs baseline: 1.0534x; 1.0534x over previous
import dataclasses
import functools
import jax
import jax.numpy as jnp
from jax import lax
from jax.experimental import pallas as pl
from jax.experimental.pallas import tpu as pltpu
from jax.experimental.pallas import tpu_sc as plsc

N_ROWS = 32768
D = 128
B = 16
L = 16
NVEC = D // L
NWC = 16
SEGC = B // 2
NR = 256
PAY = NR - 8
NBUF = 3
MAXC = 5


def _sc_segment_means(x, batch_lengths, seg_ids):
    mesh = plsc.VectorSubcoreMesh(
        core_axis_name="c", subcore_axis_name="s", num_cores=2, num_subcores=16
    )
    cp = pltpu.CompilerParams()
    if "needs_layout_passes" in pltpu.CompilerParams.__dataclass_fields__:
        cp = dataclasses.replace(cp, needs_layout_passes=False)

    @functools.partial(
        pl.kernel,
        out_type=jax.ShapeDtypeStruct((B, D), jnp.float32),
        mesh=mesh,
        scratch_types=[
            pltpu.VMEM((B,), jnp.int32),
            pltpu.VMEM((B,), jnp.int32),
            pltpu.VMEM((NR, D), jnp.float32),
            pltpu.VMEM((NR, D), jnp.float32),
            pltpu.VMEM((NR, D), jnp.float32),
            pltpu.VMEM((B, D), jnp.float32),
            pltpu.VMEM((D,), jnp.float32),
            pltpu.VMEM_SHARED((B, D), jnp.float32),
            pltpu.SemaphoreType.DMA,
            pltpu.SemaphoreType.DMA,
            pltpu.SemaphoreType.DMA,
        ],
        compiler_params=cp,
    )
    def kern(x_hbm, len_hbm, ids_hbm, out_hbm, len_vmem, idx_vmem,
             buf0, buf1, buf2, part, row_vmem, shared, sem0, sem1, sem2):
        c = lax.axis_index("c")
        s = lax.axis_index("s")

        pltpu.sync_copy(len_hbm, len_vmem)
        pltpu.sync_copy(ids_hbm, idx_vmem)
        lv = len_vmem[...]
        ends = plsc.cumsum(lv)
        lanes = lax.iota(jnp.int32, L)
        zeros_i = jnp.zeros((L,), jnp.int32)

        def lane(vec, i):
            return jnp.sum(jnp.where(lanes == i, vec, zeros_i))

        seg0 = c * SEGC
        core_lo = lane(ends, seg0 - 1)
        core_hi = lane(ends, seg0 + SEGC - 1)
        seg_end = [lane(ends, seg0 + i) for i in range(SEGC)]
        seg_len = [lane(lv, seg0 + i) for i in range(SEGC)]

        core_rows = core_hi - core_lo
        rows_per_w = (core_rows + NWC - 1) // NWC
        r0 = core_lo + s * rows_per_w
        r1 = jnp.minimum(r0 + rows_per_w, core_hi)

        zf = jnp.zeros((L,), jnp.float32)
        for i in range(B):
            for j in range(NVEC):
                part[i, pl.ds(L * j, L)] = zf

        bufs = [buf0, buf1, buf2, buf0, buf1]
        sems = [sem0, sem1, sem2, sem0, sem1]
        gstarts, aligneds, glens = [], [], []
        for k in range(MAXC):
            gstart = r0 + k * PAY
            aligned = jnp.minimum((gstart // 8) * 8, N_ROWS - NR)
            glen = jnp.minimum(PAY, r1 - gstart)
            gstarts.append(gstart)
            aligneds.append(aligned)
            glens.append(glen)

        def start_copy(k):
            pltpu.async_copy(
                x_hbm.at[pl.ds(aligneds[k], NR)], bufs[k], sems[k]
            )

        def wait_copy(k):
            pltpu.make_async_copy(
                x_hbm.at[pl.ds(aligneds[k], NR)], bufs[k], sems[k]
            ).wait()

        def process(k):
            buf = bufs[k]
            gstart, glen = gstarts[k], glens[k]
            off = gstart - aligneds[k]
            gend = gstart + glen
            for i in range(SEGC):
                r = seg0 + i
                lo = jnp.maximum(seg_end[i] - seg_len[i], gstart)
                hi = jnp.minimum(seg_end[i], gend)
                n = hi - lo

                @pl.when(n > 0)
                def _():
                    base = off + (lo - gstart)

                    def row_body(q, a):
                        p = base + q
                        return tuple(
                            a[j] + buf[p, pl.ds(L * j, L)] for j in range(NVEC)
                        )
                    accs = lax.fori_loop(
                        0, n, row_body,
                        tuple(zf for _ in range(NVEC)),
                    )
                    for j in range(NVEC):
                        part[r, pl.ds(L * j, L)] = (
                            part[r, pl.ds(L * j, L)] + accs[j]
                        )

        for k in range(NBUF):
            @pl.when(glens[k] > 0)
            def _(k=k):
                start_copy(k)

        @pl.when(s == 0)
        def _():
            pltpu.sync_copy(part, shared)
        plsc.subcore_barrier()

        for k in range(MAXC):
            @pl.when(glens[k] > 0)
            def _(k=k):
                wait_copy(k)
                process(k)
            if k + NBUF < MAXC:
                @pl.when(glens[k + NBUF] > 0)
                def _(k=k):
                    start_copy(k + NBUF)

        pltpu.sync_copy(part, shared.at[idx_vmem], add=True)
        plsc.subcore_barrier()

        @pl.when(s < SEGC)
        def _():
            r = seg0 + s
            pltpu.sync_copy(shared.at[r], row_vmem)
            den = jnp.full((L,), lane(lv, r), jnp.float32)
            for j in range(NVEC):
                row_vmem[pl.ds(L * j, L)] = row_vmem[pl.ds(L * j, L)] / den
            pltpu.sync_copy(row_vmem, out_hbm.at[r])

    return kern(x, batch_lengths, seg_ids)


def kernel(x, batch_lengths):
    lens = batch_lengths.astype(jnp.int32)
    seg_ids = jnp.arange(B, dtype=jnp.int32)
    return _sc_segment_means(x, lens, seg_ids)
